# baseline (speedup 1.0000x reference)
import functools

import jax
import jax.numpy as jnp
from jax import lax
from jax.experimental import pallas as pl
from jax.experimental.pallas import tpu as pltpu
from jax.experimental.pallas import tpu_sc as plsc

_NUM_CORES = 2
_NUM_SUBCORES = 16
_NW = _NUM_CORES * _NUM_SUBCORES


def _sc_tail_body(chunk, offset, rows, table_hbm, out_hbm, buf,
                  si0, si1, so0, so1):
    batch = out_hbm.shape[0]
    rows_per_w = rows // _NW
    n = rows_per_w // chunk
    wid = lax.axis_index("s") * _NUM_CORES + lax.axis_index("c")
    base = offset + wid * rows_per_w
    in_sems = (si0, si1)
    out_sems = (so0, so1)
    fetches = [
        pltpu.make_async_copy(
            table_hbm.at[pl.ds(base + c * chunk, chunk)],
            buf.at[c % 2], in_sems[c % 2])
        for c in range(n)
    ]
    writes = [
        [pltpu.make_async_copy(
            buf.at[c % 2],
            out_hbm.at[b, pl.ds(base + c * chunk, chunk)], out_sems[c % 2])
         for b in range(batch)]
        for c in range(n)
    ]
    fetches[0].start()
    for c in range(n):
        if c + 1 < n:
            if c >= 1:
                for w in writes[c - 1]:
                    w.wait()
            fetches[c + 1].start()
        fetches[c].wait()
        for w in writes[c]:
            w.start()
    for c in range(max(0, n - 2), n):
        for w in writes[c]:
            w.wait()


def _sc_tail_bcast(table, batch, offset):
    seq, dim = table.shape
    rows = seq - offset
    rows_per_w = rows // _NW
    chunk = rows_per_w
    while 2 * chunk * dim * 4 > 510 * 1024 or chunk % 8:
        chunk //= 2
    assert chunk % 8 == 0 and rows_per_w % chunk == 0
    mesh = plsc.VectorSubcoreMesh(
        core_axis_name="c", subcore_axis_name="s",
        num_cores=_NUM_CORES, num_subcores=_NUM_SUBCORES)
    return pl.kernel(
        functools.partial(_sc_tail_body, chunk, offset, rows), mesh=mesh,
        out_type=jax.ShapeDtypeStruct((batch, seq, dim), table.dtype),
        scratch_types=[
            pltpu.VMEM((2, chunk, dim), table.dtype),
            pltpu.SemaphoreType.DMA,
            pltpu.SemaphoreType.DMA,
            pltpu.SemaphoreType.DMA,
            pltpu.SemaphoreType.DMA,
        ],
    )(table)


def _tc_head_body(t_ref, _, o_ref):
    o_ref[...] = jnp.broadcast_to(t_ref[...][None], o_ref.shape)


def kernel(x, table):
    batch = x.shape[0]
    seq, dim = table.shape
    return _sc_tail_bcast(table, batch, offset=0)


# speedup vs baseline: 1.0546x; 1.0546x over previous
import functools

import jax
import jax.numpy as jnp
from jax import lax
from jax.experimental import pallas as pl
from jax.experimental.pallas import tpu as pltpu
from jax.experimental.pallas import tpu_sc as plsc

_NUM_CORES = 2
_NUM_SUBCORES = 16
_NW = _NUM_CORES * _NUM_SUBCORES


def _sc_tail_body(chunk, offset, rows, table_hbm, out_hbm, buf,
                  si0, si1, so0, so1):
    batch = out_hbm.shape[0]
    rows_per_w = rows // _NW
    n = rows_per_w // chunk
    wid = lax.axis_index("s") * _NUM_CORES + lax.axis_index("c")
    base = offset + wid * rows_per_w
    in_sems = (si0, si1)
    out_sems = (so0, so1)
    fetches = [
        pltpu.make_async_copy(
            table_hbm.at[pl.ds(base + c * chunk, chunk)],
            buf.at[c % 2], in_sems[c % 2])
        for c in range(n)
    ]
    writes = [
        [pltpu.make_async_copy(
            buf.at[c % 2],
            out_hbm.at[b, pl.ds(base + c * chunk, chunk)], out_sems[c % 2])
         for b in range(batch)]
        for c in range(n)
    ]
    fetches[0].start()
    for c in range(n):
        if c + 1 < n:
            if c >= 1:
                for w in writes[c - 1]:
                    w.wait()
            fetches[c + 1].start()
        fetches[c].wait()
        for w in writes[c]:
            w.start()
    for c in range(max(0, n - 2), n):
        for w in writes[c]:
            w.wait()


def _sc_tail_bcast(table, batch, offset):
    seq, dim = table.shape
    rows = seq - offset
    rows_per_w = rows // _NW
    chunk = rows_per_w
    while 2 * chunk * dim * 4 > 510 * 1024 or chunk % 8:
        chunk //= 2
    assert chunk % 8 == 0 and rows_per_w % chunk == 0
    mesh = plsc.VectorSubcoreMesh(
        core_axis_name="c", subcore_axis_name="s",
        num_cores=_NUM_CORES, num_subcores=_NUM_SUBCORES)
    return pl.kernel(
        functools.partial(_sc_tail_body, chunk, offset, rows), mesh=mesh,
        out_type=jax.ShapeDtypeStruct((batch, seq, dim), table.dtype),
        scratch_types=[
            pltpu.VMEM((2, chunk, dim), table.dtype),
            pltpu.SemaphoreType.DMA,
            pltpu.SemaphoreType.DMA,
            pltpu.SemaphoreType.DMA,
            pltpu.SemaphoreType.DMA,
        ],
    )(table)


def _tc_head_body(t_ref, _, o_ref):
    o_ref[...] = jnp.broadcast_to(t_ref[...][None], o_ref.shape)


def kernel(x, table):
    batch = x.shape[0]
    seq, dim = table.shape
    k = 5120
    bs = 512
    sc_out = _sc_tail_bcast(table, batch, offset=k)
    out = pl.pallas_call(
        _tc_head_body,
        grid=(k // bs,),
        in_specs=[
            pl.BlockSpec((bs, dim), lambda s: (s, 0)),
            pl.BlockSpec(memory_space=pl.ANY),
        ],
        out_specs=pl.BlockSpec((batch, bs, dim), lambda s: (0, s, 0)),
        out_shape=jax.ShapeDtypeStruct((batch, seq, dim), table.dtype),
        input_output_aliases={1: 0},
    )(table, sc_out)
    return out


# speedup vs baseline: 1.0614x; 1.0064x over previous
import functools

import jax
import jax.numpy as jnp
from jax import lax
from jax.experimental import pallas as pl
from jax.experimental.pallas import tpu as pltpu
from jax.experimental.pallas import tpu_sc as plsc

_NUM_CORES = 2
_NUM_SUBCORES = 16
_NW = _NUM_CORES * _NUM_SUBCORES


def _sc_tail_body(chunk, offset, rows, table_hbm, out_hbm, buf,
                  si0, si1, so0, so1):
    batch = out_hbm.shape[0]
    rows_per_w = rows // _NW
    n = rows_per_w // chunk
    wid = lax.axis_index("s") * _NUM_CORES + lax.axis_index("c")
    base = offset + wid * rows_per_w
    in_sems = (si0, si1)
    out_sems = (so0, so1)
    fetches = [
        pltpu.make_async_copy(
            table_hbm.at[pl.ds(base + c * chunk, chunk)],
            buf.at[c % 2], in_sems[c % 2])
        for c in range(n)
    ]
    writes = [
        [pltpu.make_async_copy(
            buf.at[c % 2],
            out_hbm.at[b, pl.ds(base + c * chunk, chunk)], out_sems[c % 2])
         for b in range(batch)]
        for c in range(n)
    ]
    fetches[0].start()
    for c in range(n):
        if c + 1 < n:
            if c >= 1:
                for w in writes[c - 1]:
                    w.wait()
            fetches[c + 1].start()
        fetches[c].wait()
        for w in writes[c]:
            w.start()
    for c in range(max(0, n - 2), n):
        for w in writes[c]:
            w.wait()


def _sc_tail_bcast(table, batch, offset):
    seq, dim = table.shape
    rows = seq - offset
    rows_per_w = rows // _NW
    chunk = rows_per_w
    while 2 * chunk * dim * 4 > 510 * 1024 or chunk % 8:
        chunk //= 2
    assert chunk % 8 == 0 and rows_per_w % chunk == 0
    mesh = plsc.VectorSubcoreMesh(
        core_axis_name="c", subcore_axis_name="s",
        num_cores=_NUM_CORES, num_subcores=_NUM_SUBCORES)
    return pl.kernel(
        functools.partial(_sc_tail_body, chunk, offset, rows), mesh=mesh,
        out_type=jax.ShapeDtypeStruct((batch, seq, dim), table.dtype),
        scratch_types=[
            pltpu.VMEM((2, chunk, dim), table.dtype),
            pltpu.SemaphoreType.DMA,
            pltpu.SemaphoreType.DMA,
            pltpu.SemaphoreType.DMA,
            pltpu.SemaphoreType.DMA,
        ],
    )(table)


def _tc_head_body(t_ref, _, o_ref):
    o_ref[...] = jnp.broadcast_to(t_ref[...][None], o_ref.shape)


def kernel(x, table):
    batch = x.shape[0]
    seq, dim = table.shape
    k = 5120
    bs = 1024
    sc_out = _sc_tail_bcast(table, batch, offset=k)
    out = pl.pallas_call(
        _tc_head_body,
        grid=(k // bs,),
        in_specs=[
            pl.BlockSpec((bs, dim), lambda s: (s, 0)),
            pl.BlockSpec(memory_space=pl.ANY),
        ],
        out_specs=pl.BlockSpec((batch, bs, dim), lambda s: (0, s, 0)),
        out_shape=jax.ShapeDtypeStruct((batch, seq, dim), table.dtype),
        input_output_aliases={1: 0},
    )(table, sc_out)
    return out
